# baseline (speedup 1.0000x reference)
import jax
import jax.numpy as jnp
from jax import lax
from jax.experimental import pallas as pl
from jax.experimental.pallas import tpu as pltpu
from jax.experimental.pallas import tpu_sc as plsc

_N = 100000
_D = 128
_B = 64
_G = 4

_L = 16
_NW = 32
_RPW = 3200
_TAIL = _N - (_NW - 1) * _RPW
_BR = 400
_NB = _RPW // _BR
_DEPTH = 2


def _sc_body(gran_hbm, batch_hbm, state_hbm, pae_hbm, nac_hbm,
             batch_v, state_v, eidx_r, nidx_r, c15_v, erows_r, nrows_r,
             pae_v, nac_v, *sems):
    esems, nsems = sems[:_DEPTH], sems[_DEPTH:]
    cid = lax.axis_index("c")
    sid = lax.axis_index("s")
    wid = sid * 2 + cid
    base = wid * _RPW
    is_tail = wid == _NW - 1

    pltpu.sync_copy(state_hbm, state_v)

    @pl.when(jnp.logical_not(is_tail))
    def _():
        pltpu.sync_copy(batch_hbm.at[pl.ds(base, _RPW)], batch_v)

    @pl.when(is_tail)
    def _():
        pltpu.sync_copy(batch_hbm.at[pl.ds(base, _TAIL)],
                        batch_v.at[pl.ds(0, _TAIL)])

    def build_and_fire(g):
        slot = g % _DEPTH

        def build(k, carry):
            lanes = lax.broadcasted_iota(jnp.int32, (_L,), 0)
            r_loc = g * _BR + k * _L + lanes
            r_cl = jnp.minimum(base + r_loc, _N - 1)
            gr = r_cl * _G
            eidx_r[slot, pl.ds(k * _L, _L)] = gr
            nidx_r[slot, pl.ds(k * _L, _L)] = gr
            return carry

        lax.fori_loop(0, _BR // _L, build, 0)
        eh = pltpu.async_copy(gran_hbm.at[eidx_r.at[slot]],
                              erows_r.at[slot], esems[slot])
        nh = pltpu.async_copy(gran_hbm.at[nidx_r.at[slot]],
                              nrows_r.at[slot], nsems[slot])
        return eh, nh

    def consume(g, eh, nh):
        eh.wait()
        nh.wait()

    inflight = []
    for g in range(_NB):
        inflight.append(build_and_fire(g))
        if g >= _DEPTH - 1:
            gc = g - (_DEPTH - 1)
            consume(gc, *inflight[gc])
    for gc in range(_NB - _DEPTH + 1, _NB):
        consume(gc, *inflight[gc])

    @pl.when(jnp.logical_not(is_tail))
    def _():
        pltpu.sync_copy(pae_v, pae_hbm.at[pl.ds(base, _RPW)])
        pltpu.sync_copy(nac_v, nac_hbm.at[pl.ds(base, _RPW), :])

    @pl.when(is_tail)
    def _():
        pltpu.sync_copy(pae_v.at[pl.ds(0, _TAIL)],
                        pae_hbm.at[pl.ds(base, _TAIL)])
        pltpu.sync_copy(nac_v.at[pl.ds(0, _TAIL), :],
                        nac_hbm.at[pl.ds(base, _TAIL), :])


def _make_sc_call():
    mesh = plsc.VectorSubcoreMesh(core_axis_name="c", subcore_axis_name="s")
    return pl.kernel(
        _sc_body,
        mesh=mesh,
        compiler_params=pltpu.CompilerParams(
            needs_layout_passes=False, use_tc_tiling_on_sc=False,
            skip_device_barrier=True, disable_bounds_checks=True,
            disable_semaphore_checks=True),
        out_type=(
            jax.ShapeDtypeStruct((_N,), jnp.float32),
            jax.ShapeDtypeStruct((_N, 3), jnp.float32),
        ),
        scratch_types=[
            pltpu.VMEM((_RPW,), jnp.int32),
            pltpu.VMEM((128,), jnp.int32),
            pltpu.VMEM((_DEPTH, _BR), jnp.int32),
            pltpu.VMEM((_DEPTH, _BR), jnp.int32),
            pltpu.VMEM((_RPW,), jnp.int32),
            pltpu.VMEM((_DEPTH, _BR, 2 * _L), jnp.float32),
            pltpu.VMEM((_DEPTH, _BR, 2 * _L), jnp.float32),
            pltpu.VMEM((_RPW,), jnp.float32),
            pltpu.VMEM((_RPW, 3), jnp.float32),
        ] + [pltpu.SemaphoreType.DMA] * (2 * _DEPTH),
    )


def kernel(node_features, batch, state):
    gran = node_features.reshape(_N * _G, 2 * _L)
    batch_i = batch.astype(jnp.int32)
    state_pad = jnp.concatenate(
        [state.astype(jnp.int32), jnp.zeros((128 - _B,), jnp.int32)])
    pae, nac = _make_sc_call()(gran, batch_i, state_pad)
    return (pae.reshape(_N, 1), nac)


# speedup vs baseline: 1.6028x; 1.6028x over previous
import jax
import jax.numpy as jnp
from jax import lax
from jax.experimental import pallas as pl
from jax.experimental.pallas import tpu as pltpu
from jax.experimental.pallas import tpu_sc as plsc

_N = 100000
_D = 128
_B = 64
_G = 8

_L = 16
_NW = 32
_RPW = 3200
_TAIL = _N - (_NW - 1) * _RPW
_BR = 128
_BL = _BR * _G
_NB = _RPW // _BR
_DEPTH = 3


def _sc_body(gran_hbm, batch_hbm, state_hbm, pae_hbm, nac_hbm,
             batch_v, state_v, slab_r, pae_v, nac_v, *sems):
    cid = lax.axis_index("c")
    sid = lax.axis_index("s")
    wid = sid * 2 + cid
    base = wid * _RPW
    is_tail = wid == _NW - 1

    pltpu.sync_copy(state_hbm, state_v)

    @pl.when(jnp.logical_not(is_tail))
    def _():
        pltpu.sync_copy(batch_hbm.at[pl.ds(base, _RPW)], batch_v)

    @pl.when(is_tail)
    def _():
        pltpu.sync_copy(batch_hbm.at[pl.ds(base, _TAIL)],
                        batch_v.at[pl.ds(0, _TAIL)])

    def fire(g):
        slot = g % _DEPTH
        src = jnp.minimum((base + g * _BR) * _G, _N * _G - _BL)
        return pltpu.async_copy(gran_hbm.at[pl.ds(src, _BL)],
                                slab_r.at[slot], sems[slot])

    def consume(g, h):
        slot = g % _DEPTH
        h.wait()

        def extract(k, carry):
            lanes = lax.broadcasted_iota(jnp.int32, (_L,), 0)
            r_ib = k * _L + lanes
            r_loc = g * _BR + r_ib
            b = jnp.clip(batch_v[pl.ds(g * _BR + k * _L, _L)], 0, _B - 1)
            c = plsc.load_gather(state_v, [b])
            pae_v[pl.ds(g * _BR + k * _L, _L)] = plsc.load_gather(
                slab_r.at[slot], [r_ib * _G + (c >> 4), c & (_L - 1)])
            for j in range(3):
                vj = plsc.load_gather(slab_r.at[slot],
                                      [r_ib * _G, lanes * 0 + (2 + j)])
                plsc.store_scatter(nac_v, [r_loc, lanes * 0 + j], vj)
            return carry

        lax.fori_loop(0, _BR // _L, extract, 0)

    inflight = []
    for g in range(_NB):
        inflight.append(fire(g))
        if g >= _DEPTH - 1:
            gc = g - (_DEPTH - 1)
            consume(gc, inflight[gc])
    for gc in range(_NB - _DEPTH + 1, _NB):
        consume(gc, inflight[gc])

    @pl.when(jnp.logical_not(is_tail))
    def _():
        pltpu.sync_copy(pae_v, pae_hbm.at[pl.ds(base, _RPW)])
        pltpu.sync_copy(nac_v, nac_hbm.at[pl.ds(base, _RPW), :])

    @pl.when(is_tail)
    def _():
        pltpu.sync_copy(pae_v.at[pl.ds(0, _TAIL)],
                        pae_hbm.at[pl.ds(base, _TAIL)])
        pltpu.sync_copy(nac_v.at[pl.ds(0, _TAIL), :],
                        nac_hbm.at[pl.ds(base, _TAIL), :])


def _make_sc_call():
    mesh = plsc.VectorSubcoreMesh(core_axis_name="c", subcore_axis_name="s")
    return pl.kernel(
        _sc_body,
        mesh=mesh,
        compiler_params=pltpu.CompilerParams(
            needs_layout_passes=False, use_tc_tiling_on_sc=False,
            skip_device_barrier=True, disable_bounds_checks=True,
            disable_semaphore_checks=True),
        out_type=(
            jax.ShapeDtypeStruct((_N,), jnp.float32),
            jax.ShapeDtypeStruct((_N, 3), jnp.float32),
        ),
        scratch_types=[
            pltpu.VMEM((_RPW,), jnp.int32),
            pltpu.VMEM((128,), jnp.int32),
            pltpu.VMEM((_DEPTH, _BL, _L), jnp.float32),
            pltpu.VMEM((_RPW,), jnp.float32),
            pltpu.VMEM((_RPW, 3), jnp.float32),
        ] + [pltpu.SemaphoreType.DMA] * _DEPTH,
    )


def kernel(node_features, batch, state):
    gran = node_features.reshape(_N * _G, _L)
    batch_i = batch.astype(jnp.int32)
    state_pad = jnp.concatenate(
        [state.astype(jnp.int32), jnp.zeros((128 - _B,), jnp.int32)])
    pae, nac = _make_sc_call()(gran, batch_i, state_pad)
    return (pae.reshape(_N, 1), nac)
